# baseline (speedup 1.0000x reference)
import functools

import jax
import jax.numpy as jnp
from jax import lax
from jax.experimental import pallas as pl
from jax.experimental.pallas import tpu as pltpu
from jax.experimental.pallas import tpu_sc as plsc

N = 10000
F = 128
E = 320000
NPAD = 10240
NC = 2
NS = 16
NW = NC * NS
K = 64
CHT = (-(-E // (NW * K)) + 7) // 8 * 8
EPAD = NW * K * CHT
RPT = NPAD // NS
RB = 2048
GRID = NPAD // RB


def _deg_body(dst_hbm, out_hbm, dstb, hist):
    cid = lax.axis_index("c")
    sid = lax.axis_index("s")
    wid = sid * NC + cid
    pltpu.sync_copy(dst_hbm.at[pl.ds(wid * CHT, CHT)], dstb)
    zero16 = jnp.zeros((16,), jnp.float32)

    def zbody(i, carry):
        hist[pl.ds(i * 16, 16)] = zero16
        return carry

    lax.fori_loop(0, NPAD // 16, zbody, 0)

    ones16 = jnp.ones((16,), jnp.float32)

    def ebody(i, carry):
        j = i // (K // 16)
        g = i % (K // 16)
        idx = dstb[j, pl.ds(g * 16, 16)]
        plsc.addupdate_scatter(hist, [idx], ones16)
        return carry

    lax.fori_loop(0, CHT * (K // 16), ebody, 0)
    pltpu.sync_copy(hist, out_hbm.at[wid, 0])


NBUF = 4
CHT_A = 288
CHT_B = (EPAD // K - 16 * CHT_A) // 16


def _seg_pipeline(tbl_hbm, eidx_hbm, acc, eb, rows, gsem, ssem, isem,
                  base, ng):
    pltpu.sync_copy(eidx_hbm.at[pl.ds(base, NBUF)], eb.at[0])
    pltpu.async_copy(eidx_hbm.at[pl.ds(base + NBUF, NBUF)], eb.at[1], isem)
    for b in range(NBUF):
        pltpu.async_copy(tbl_hbm.at[eb.at[0, b, 0]], rows[b], gsem[b])

    def group(g, carry):
        slot = lax.rem(g, 2)
        nslot = 1 - slot
        for b in range(NBUF):
            pltpu.make_async_copy(
                tbl_hbm.at[eb.at[slot, b, 0]], rows[b], gsem[b]).wait()
            pltpu.async_copy(
                rows[b], acc.at[eb.at[slot, b, 1]], ssem[b], add=True)

        @pl.when(g + 1 < ng)
        def _():
            pltpu.make_async_copy(
                eidx_hbm.at[pl.ds(base, NBUF)], eb.at[nslot], isem).wait()
            for b in range(NBUF):
                pltpu.make_async_copy(
                    rows[b], acc.at[eb.at[slot, b, 1]], ssem[b]).wait()
                pltpu.async_copy(
                    tbl_hbm.at[eb.at[nslot, b, 0]], rows[b], gsem[b])

        @pl.when(g + 2 < ng)
        def _():
            pltpu.async_copy(
                eidx_hbm.at[pl.ds(base + (g + 2) * NBUF, NBUF)],
                eb.at[slot], isem)
        return carry

    lax.fori_loop(0, ng, group, 0)
    for b in range(NBUF):
        pltpu.make_async_copy(rows[b], acc.at[eb.at[0, b, 1]], ssem[b]).wait()


def _segsum_body(tbl_hbm, eidx_hbm, zeros_hbm, out_hbm,
                 eb, r0, r1, r2, r3, acc,
                 g0, g1, g2, g3, s0, s1, s2, s3, isem):
    rows = [r0, r1, r2, r3]
    gsem = [g0, g1, g2, g3]
    ssem = [s0, s1, s2, s3]
    cid = lax.axis_index("c")
    sid = lax.axis_index("s")
    pltpu.sync_copy(zeros_hbm.at[pl.ds(sid * RPT, RPT)],
                    acc.at[pl.ds(sid * RPT, RPT)])
    plsc.subcore_barrier()

    @pl.when(cid == 0)
    def _():
        _seg_pipeline(tbl_hbm, eidx_hbm, acc, eb, rows, gsem, ssem, isem,
                      sid * CHT_A, CHT_A // NBUF)

    if CHT_B > 0:
        @pl.when(cid == 1)
        def _():
            _seg_pipeline(tbl_hbm, eidx_hbm, acc, eb, rows, gsem, ssem, isem,
                          16 * CHT_A + sid * CHT_B, CHT_B // NBUF)

    plsc.subcore_barrier()
    pltpu.sync_copy(acc.at[pl.ds(sid * RPT, RPT)],
                    out_hbm.at[cid, pl.ds(sid * RPT, RPT)])


@functools.lru_cache(maxsize=None)
def _sc_kernels():
    mesh = plsc.VectorSubcoreMesh(
        core_axis_name="c", subcore_axis_name="s",
        num_cores=NC, num_subcores=NS)
    cparams = pltpu.CompilerParams(needs_layout_passes=False)
    deg = pl.kernel(
        _deg_body,
        out_type=jax.ShapeDtypeStruct((NW, 1, NPAD), jnp.float32),
        mesh=mesh,
        compiler_params=cparams,
        scratch_types=[
            pltpu.VMEM((CHT, K), jnp.int32),
            pltpu.VMEM((NPAD,), jnp.float32),
        ],
    )
    segsum = pl.kernel(
        _segsum_body,
        out_type=jax.ShapeDtypeStruct((NC, NPAD, F), jnp.float32),
        mesh=mesh,
        compiler_params=cparams,
        scratch_types=(
            [pltpu.VMEM((2, NBUF, 2, K), jnp.int32)]
            + [pltpu.VMEM((K, F), jnp.float32) for _ in range(NBUF)]
            + [pltpu.VMEM_SHARED((NPAD, F), jnp.float32)]
            + [pltpu.SemaphoreType.DMA for _ in range(2 * NBUF + 1)]
        ),
    )
    return deg, segsum


def _mlp_body(x_ref, w1_ref, b1_ref, w2_ref, b2_ref, wr_ref, br_ref,
              h_ref, res_ref):
    x = x_ref[...]
    h1 = jnp.maximum(jnp.dot(x, w1_ref[...],
                             preferred_element_type=jnp.float32)
                     + b1_ref[...], 0.0)
    h = jnp.maximum(jnp.dot(h1, w2_ref[...],
                            preferred_element_type=jnp.float32)
                    + b2_ref[...], 0.0)
    h_ref[...] = h
    res_ref[...] = jnp.dot(h, wr_ref[...],
                           preferred_element_type=jnp.float32) + br_ref[...]


def _scale_body(degp_ref, h_ref, dinv_ref, s_ref):
    deg = jnp.sum(degp_ref[...], axis=0)
    dinv = lax.rsqrt(jnp.maximum(deg, 1.0))
    dinv_ref[...] = dinv
    s_ref[...] = h_ref[...] * dinv[:, None]


def _update_body(part_ref, h_ref, dinv_ref, f_ref, s_ref):
    agg = part_ref[0] + part_ref[1]
    dinv = dinv_ref[...][:, None]
    f = h_ref[...] - agg * dinv
    f_ref[...] = f
    s_ref[...] = f * dinv


def _final_body(part_ref, h_ref, f1_ref, dinv_ref, res_ref,
                wa_ref, ba_ref, wf1_ref, bf1_ref, wf2_ref, bf2_ref,
                w3_ref, b3_ref, w4_ref, b4_ref, out_ref):
    dinv = dinv_ref[...][:, None]
    f0 = h_ref[...]
    f1 = f1_ref[...]
    f2 = f1 - (part_ref[0] + part_ref[1]) * dinv
    br0 = 3.0 * f0 - 3.0 * f1 + 0.75 * f2
    br1 = 3.0 * f1 - 1.5 * f2
    br2 = 0.75 * f2
    wa = wa_ref[...]
    ba = ba_ref[...]
    s0 = jnp.dot(br0, wa, preferred_element_type=jnp.float32) + ba
    s1 = jnp.dot(br1, wa, preferred_element_type=jnp.float32) + ba
    s2 = jnp.dot(br2, wa, preferred_element_type=jnp.float32) + ba
    m = jnp.maximum(jnp.maximum(s0, s1), s2)
    e0 = jnp.exp(s0 - m)
    e1 = jnp.exp(s1 - m)
    e2 = jnp.exp(s2 - m)
    den = e0 + e1 + e2
    attn = (e0 * br0 + e1 * br1 + e2 * br2) / den
    mean = (br0 + br1 + br2) * (1.0 / 3.0)
    z = jnp.maximum(
        jnp.dot(attn, wf1_ref[0:F, :], preferred_element_type=jnp.float32)
        + jnp.dot(mean, wf1_ref[F:2 * F, :], preferred_element_type=jnp.float32)
        + bf1_ref[...], 0.0)
    logit_fw = jnp.dot(z, wf2_ref[...],
                       preferred_element_type=jnp.float32) + bf2_ref[...]
    fw = 1.0 / (1.0 + jnp.exp(-logit_fw))
    fused = 0.1 * fw * attn + (1.0 - fw) * mean + 0.8 * res_ref[...]
    h3 = jnp.maximum(jnp.dot(fused, w3_ref[...],
                             preferred_element_type=jnp.float32)
                     + b3_ref[...], 0.0)
    out_ref[...] = jnp.dot(h3, w4_ref[...],
                           preferred_element_type=jnp.float32) + b4_ref[...]


def _row_spec():
    return pl.BlockSpec((RB, F), lambda i: (i, 0))


def _full_spec(shape):
    nd = len(shape)
    return pl.BlockSpec(shape, lambda i: (0,) * nd)


def _mlp_call(x, W1, b1, W2, b2, Wres, bres):
    return pl.pallas_call(
        _mlp_body,
        grid=(GRID,),
        in_specs=[_row_spec(), _full_spec((F, F)), _full_spec((1, F)),
                  _full_spec((F, F)), _full_spec((1, F)),
                  _full_spec((F, F)), _full_spec((1, F))],
        out_specs=[_row_spec(), _row_spec()],
        out_shape=[jax.ShapeDtypeStruct((NPAD, F), jnp.float32),
                   jax.ShapeDtypeStruct((NPAD, F), jnp.float32)],
    )(x, W1, b1, W2, b2, Wres, bres)


def _scale_call(degp, h):
    return pl.pallas_call(
        _scale_body,
        grid=(GRID,),
        in_specs=[pl.BlockSpec((NW, RB), lambda i: (0, i)), _row_spec()],
        out_specs=[pl.BlockSpec((RB,), lambda i: (i,)), _row_spec()],
        out_shape=[jax.ShapeDtypeStruct((NPAD,), jnp.float32),
                   jax.ShapeDtypeStruct((NPAD, F), jnp.float32)],
    )(degp, h)


def _update_call(part, h, dinv):
    return pl.pallas_call(
        _update_body,
        grid=(GRID,),
        in_specs=[pl.BlockSpec((NC, RB, F), lambda i: (0, i, 0)),
                  _row_spec(), pl.BlockSpec((RB,), lambda i: (i,))],
        out_specs=[_row_spec(), _row_spec()],
        out_shape=[jax.ShapeDtypeStruct((NPAD, F), jnp.float32),
                   jax.ShapeDtypeStruct((NPAD, F), jnp.float32)],
    )(part, h, dinv)


def _final_call(part, h, f1, dinv, res,
                Wattn, battn, Wf1, bf1, Wf2, bf2, W3, b3, W4, b4):
    return pl.pallas_call(
        _final_body,
        grid=(GRID,),
        in_specs=[pl.BlockSpec((NC, RB, F), lambda i: (0, i, 0)),
                  _row_spec(), _row_spec(),
                  pl.BlockSpec((RB,), lambda i: (i,)), _row_spec(),
                  _full_spec((F, 1)), _full_spec((1, 1)),
                  _full_spec((2 * F, F)), _full_spec((1, F)),
                  _full_spec((F, 1)), _full_spec((1, 1)),
                  _full_spec((F, F)), _full_spec((1, F)),
                  _full_spec((F, 2)), _full_spec((1, 2))],
        out_specs=pl.BlockSpec((RB, 2), lambda i: (i, 0)),
        out_shape=jax.ShapeDtypeStruct((NPAD, 2), jnp.float32),
    )(part, h, f1, dinv, res,
      Wattn, battn, Wf1, bf1, Wf2, bf2, W3, b3, W4, b4)


def kernel(in_feat, edge_index, W1, b1, W2, b2, Wres, bres, Wattn, battn,
           Wf1, bf1, Wf2, bf2, W3, b3, W4, b4):
    src = edge_index[0].astype(jnp.int32)
    dst = edge_index[1].astype(jnp.int32)
    npadE = EPAD - E
    src_p = jnp.concatenate(
        [src, jnp.zeros((npadE,), jnp.int32)]).reshape(EPAD // K, K)
    dst_p = jnp.concatenate(
        [dst, jnp.full((npadE,), N, jnp.int32)]).reshape(EPAD // K, K)
    eidx = jnp.stack([src_p, dst_p], axis=1)
    x_p = jnp.pad(in_feat, ((0, NPAD - N), (0, 0)))
    zeros_tbl = jnp.zeros((NPAD, F), jnp.float32)

    b1r = b1.reshape(1, F)
    b2r = b2.reshape(1, F)
    bresr = bres.reshape(1, F)
    battnr = battn.reshape(1, 1)
    bf1r = bf1.reshape(1, F)
    bf2r = bf2.reshape(1, 1)
    b3r = b3.reshape(1, F)
    b4r = b4.reshape(1, 2)

    _deg_kernel, _segsum_kernel = _sc_kernels()
    deg_part = _deg_kernel(dst_p).reshape(NW, NPAD)
    h_pre, res = _mlp_call(x_p, W1, b1r, W2, b2r, Wres, bresr)
    dinv, s1 = _scale_call(deg_part, h_pre)
    part1 = _segsum_kernel(s1, eidx, zeros_tbl)
    f1, s2 = _update_call(part1, h_pre, dinv)
    part2 = _segsum_kernel(s2, eidx, zeros_tbl)
    logits_p = _final_call(part2, h_pre, f1, dinv, res,
                           Wattn, battnr, Wf1, bf1r, Wf2, bf2r,
                           W3, b3r, W4, b4r)
    return logits_p[:N]


# speedup vs baseline: 1.0119x; 1.0119x over previous
import functools

import jax
import jax.numpy as jnp
from jax import lax
from jax.experimental import pallas as pl
from jax.experimental.pallas import tpu as pltpu
from jax.experimental.pallas import tpu_sc as plsc

N = 10000
F = 128
E = 320000
NPAD = 10240
NC = 2
NS = 16
NW = NC * NS
K = 64
CHT = (-(-E // (NW * K)) + 7) // 8 * 8
EPAD = NW * K * CHT
RPT = NPAD // NS
RB = 2048
GRID = NPAD // RB


def _deg_body(dst_hbm, out_hbm, dstb, hist):
    cid = lax.axis_index("c")
    sid = lax.axis_index("s")
    wid = sid * NC + cid
    pltpu.sync_copy(dst_hbm.at[pl.ds(wid * CHT, CHT)], dstb)
    zero16 = jnp.zeros((16,), jnp.float32)

    def zbody(i, carry):
        hist[pl.ds(i * 16, 16)] = zero16
        return carry

    lax.fori_loop(0, NPAD // 16, zbody, 0)

    ones16 = jnp.ones((16,), jnp.float32)

    def ebody(i, carry):
        j = i // (K // 16)
        g = i % (K // 16)
        idx = dstb[j, pl.ds(g * 16, 16)]
        plsc.addupdate_scatter(hist, [idx], ones16)
        return carry

    lax.fori_loop(0, CHT * (K // 16), ebody, 0)
    pltpu.sync_copy(hist, out_hbm.at[wid, 0])


NBUF = 4
CHT_A = 296
CHT_B = (EPAD // K - 16 * CHT_A) // 16


def _seg_pipeline(tbl_hbm, eidx_hbm, acc, eb, rows, gsem, ssem, isem,
                  base, ng):
    pltpu.sync_copy(eidx_hbm.at[pl.ds(base, NBUF)], eb.at[0])
    pltpu.async_copy(eidx_hbm.at[pl.ds(base + NBUF, NBUF)], eb.at[1], isem)
    for b in range(NBUF):
        pltpu.async_copy(tbl_hbm.at[eb.at[0, b, 0]], rows[b], gsem[b])

    def group(g, carry):
        slot = lax.rem(g, 2)
        nslot = 1 - slot
        for b in range(NBUF):
            pltpu.make_async_copy(
                tbl_hbm.at[eb.at[slot, b, 0]], rows[b], gsem[b]).wait()
            pltpu.async_copy(
                rows[b], acc.at[eb.at[slot, b, 1]], ssem[b], add=True)

        @pl.when(g + 1 < ng)
        def _():
            pltpu.make_async_copy(
                eidx_hbm.at[pl.ds(base, NBUF)], eb.at[nslot], isem).wait()
            for b in range(NBUF):
                pltpu.make_async_copy(
                    rows[b], acc.at[eb.at[slot, b, 1]], ssem[b]).wait()
                pltpu.async_copy(
                    tbl_hbm.at[eb.at[nslot, b, 0]], rows[b], gsem[b])

        @pl.when(g + 2 < ng)
        def _():
            pltpu.async_copy(
                eidx_hbm.at[pl.ds(base + (g + 2) * NBUF, NBUF)],
                eb.at[slot], isem)
        return carry

    lax.fori_loop(0, ng, group, 0)
    for b in range(NBUF):
        pltpu.make_async_copy(rows[b], acc.at[eb.at[0, b, 1]], ssem[b]).wait()


def _segsum_body(tbl_hbm, eidx_hbm, zeros_hbm, out_hbm,
                 eb, r0, r1, r2, r3, acc,
                 g0, g1, g2, g3, s0, s1, s2, s3, isem):
    rows = [r0, r1, r2, r3]
    gsem = [g0, g1, g2, g3]
    ssem = [s0, s1, s2, s3]
    cid = lax.axis_index("c")
    sid = lax.axis_index("s")
    pltpu.sync_copy(zeros_hbm.at[pl.ds(sid * RPT, RPT)],
                    acc.at[pl.ds(sid * RPT, RPT)])
    plsc.subcore_barrier()

    @pl.when(cid == 0)
    def _():
        _seg_pipeline(tbl_hbm, eidx_hbm, acc, eb, rows, gsem, ssem, isem,
                      sid * CHT_A, CHT_A // NBUF)

    if CHT_B > 0:
        @pl.when(cid == 1)
        def _():
            _seg_pipeline(tbl_hbm, eidx_hbm, acc, eb, rows, gsem, ssem, isem,
                          16 * CHT_A + sid * CHT_B, CHT_B // NBUF)

    plsc.subcore_barrier()
    pltpu.sync_copy(acc.at[pl.ds(sid * RPT, RPT)],
                    out_hbm.at[cid, pl.ds(sid * RPT, RPT)])


@functools.lru_cache(maxsize=None)
def _sc_kernels():
    mesh = plsc.VectorSubcoreMesh(
        core_axis_name="c", subcore_axis_name="s",
        num_cores=NC, num_subcores=NS)
    cparams = pltpu.CompilerParams(needs_layout_passes=False)
    deg = pl.kernel(
        _deg_body,
        out_type=jax.ShapeDtypeStruct((NW, 1, NPAD), jnp.float32),
        mesh=mesh,
        compiler_params=cparams,
        scratch_types=[
            pltpu.VMEM((CHT, K), jnp.int32),
            pltpu.VMEM((NPAD,), jnp.float32),
        ],
    )
    segsum = pl.kernel(
        _segsum_body,
        out_type=jax.ShapeDtypeStruct((NC, NPAD, F), jnp.float32),
        mesh=mesh,
        compiler_params=cparams,
        scratch_types=(
            [pltpu.VMEM((2, NBUF, 2, K), jnp.int32)]
            + [pltpu.VMEM((K, F), jnp.float32) for _ in range(NBUF)]
            + [pltpu.VMEM_SHARED((NPAD, F), jnp.float32)]
            + [pltpu.SemaphoreType.DMA for _ in range(2 * NBUF + 1)]
        ),
    )
    return deg, segsum


def _mlp_body(x_ref, w1_ref, b1_ref, w2_ref, b2_ref, wr_ref, br_ref,
              h_ref, res_ref):
    x = x_ref[...]
    h1 = jnp.maximum(jnp.dot(x, w1_ref[...],
                             preferred_element_type=jnp.float32)
                     + b1_ref[...], 0.0)
    h = jnp.maximum(jnp.dot(h1, w2_ref[...],
                            preferred_element_type=jnp.float32)
                    + b2_ref[...], 0.0)
    h_ref[...] = h
    res_ref[...] = jnp.dot(h, wr_ref[...],
                           preferred_element_type=jnp.float32) + br_ref[...]


def _scale_body(degp_ref, h_ref, dinv_ref, s_ref):
    deg = jnp.sum(degp_ref[...], axis=0)
    dinv = lax.rsqrt(jnp.maximum(deg, 1.0))
    dinv_ref[...] = dinv
    s_ref[...] = h_ref[...] * dinv[:, None]


def _update_body(part_ref, h_ref, dinv_ref, f_ref, s_ref):
    agg = part_ref[0] + part_ref[1]
    dinv = dinv_ref[...][:, None]
    f = h_ref[...] - agg * dinv
    f_ref[...] = f
    s_ref[...] = f * dinv


def _final_body(part_ref, h_ref, f1_ref, dinv_ref, res_ref,
                wa_ref, ba_ref, wf1_ref, bf1_ref, wf2_ref, bf2_ref,
                w3_ref, b3_ref, w4_ref, b4_ref, out_ref):
    dinv = dinv_ref[...][:, None]
    f0 = h_ref[...]
    f1 = f1_ref[...]
    f2 = f1 - (part_ref[0] + part_ref[1]) * dinv
    br0 = 3.0 * f0 - 3.0 * f1 + 0.75 * f2
    br1 = 3.0 * f1 - 1.5 * f2
    br2 = 0.75 * f2
    wa = wa_ref[...]
    ba = ba_ref[...]
    s0 = jnp.dot(br0, wa, preferred_element_type=jnp.float32) + ba
    s1 = jnp.dot(br1, wa, preferred_element_type=jnp.float32) + ba
    s2 = jnp.dot(br2, wa, preferred_element_type=jnp.float32) + ba
    m = jnp.maximum(jnp.maximum(s0, s1), s2)
    e0 = jnp.exp(s0 - m)
    e1 = jnp.exp(s1 - m)
    e2 = jnp.exp(s2 - m)
    den = e0 + e1 + e2
    attn = (e0 * br0 + e1 * br1 + e2 * br2) / den
    mean = (br0 + br1 + br2) * (1.0 / 3.0)
    z = jnp.maximum(
        jnp.dot(attn, wf1_ref[0:F, :], preferred_element_type=jnp.float32)
        + jnp.dot(mean, wf1_ref[F:2 * F, :], preferred_element_type=jnp.float32)
        + bf1_ref[...], 0.0)
    logit_fw = jnp.dot(z, wf2_ref[...],
                       preferred_element_type=jnp.float32) + bf2_ref[...]
    fw = 1.0 / (1.0 + jnp.exp(-logit_fw))
    fused = 0.1 * fw * attn + (1.0 - fw) * mean + 0.8 * res_ref[...]
    h3 = jnp.maximum(jnp.dot(fused, w3_ref[...],
                             preferred_element_type=jnp.float32)
                     + b3_ref[...], 0.0)
    out_ref[...] = jnp.dot(h3, w4_ref[...],
                           preferred_element_type=jnp.float32) + b4_ref[...]


def _row_spec():
    return pl.BlockSpec((RB, F), lambda i: (i, 0))


def _full_spec(shape):
    nd = len(shape)
    return pl.BlockSpec(shape, lambda i: (0,) * nd)


def _mlp_call(x, W1, b1, W2, b2, Wres, bres):
    return pl.pallas_call(
        _mlp_body,
        grid=(GRID,),
        in_specs=[_row_spec(), _full_spec((F, F)), _full_spec((1, F)),
                  _full_spec((F, F)), _full_spec((1, F)),
                  _full_spec((F, F)), _full_spec((1, F))],
        out_specs=[_row_spec(), _row_spec()],
        out_shape=[jax.ShapeDtypeStruct((NPAD, F), jnp.float32),
                   jax.ShapeDtypeStruct((NPAD, F), jnp.float32)],
    )(x, W1, b1, W2, b2, Wres, bres)


def _scale_call(degp, h):
    return pl.pallas_call(
        _scale_body,
        grid=(GRID,),
        in_specs=[pl.BlockSpec((NW, RB), lambda i: (0, i)), _row_spec()],
        out_specs=[pl.BlockSpec((RB,), lambda i: (i,)), _row_spec()],
        out_shape=[jax.ShapeDtypeStruct((NPAD,), jnp.float32),
                   jax.ShapeDtypeStruct((NPAD, F), jnp.float32)],
    )(degp, h)


def _update_call(part, h, dinv):
    return pl.pallas_call(
        _update_body,
        grid=(GRID,),
        in_specs=[pl.BlockSpec((NC, RB, F), lambda i: (0, i, 0)),
                  _row_spec(), pl.BlockSpec((RB,), lambda i: (i,))],
        out_specs=[_row_spec(), _row_spec()],
        out_shape=[jax.ShapeDtypeStruct((NPAD, F), jnp.float32),
                   jax.ShapeDtypeStruct((NPAD, F), jnp.float32)],
    )(part, h, dinv)


def _final_call(part, h, f1, dinv, res,
                Wattn, battn, Wf1, bf1, Wf2, bf2, W3, b3, W4, b4):
    return pl.pallas_call(
        _final_body,
        grid=(GRID,),
        in_specs=[pl.BlockSpec((NC, RB, F), lambda i: (0, i, 0)),
                  _row_spec(), _row_spec(),
                  pl.BlockSpec((RB,), lambda i: (i,)), _row_spec(),
                  _full_spec((F, 1)), _full_spec((1, 1)),
                  _full_spec((2 * F, F)), _full_spec((1, F)),
                  _full_spec((F, 1)), _full_spec((1, 1)),
                  _full_spec((F, F)), _full_spec((1, F)),
                  _full_spec((F, 2)), _full_spec((1, 2))],
        out_specs=pl.BlockSpec((RB, 2), lambda i: (i, 0)),
        out_shape=jax.ShapeDtypeStruct((NPAD, 2), jnp.float32),
    )(part, h, f1, dinv, res,
      Wattn, battn, Wf1, bf1, Wf2, bf2, W3, b3, W4, b4)


def kernel(in_feat, edge_index, W1, b1, W2, b2, Wres, bres, Wattn, battn,
           Wf1, bf1, Wf2, bf2, W3, b3, W4, b4):
    src = edge_index[0].astype(jnp.int32)
    dst = edge_index[1].astype(jnp.int32)
    npadE = EPAD - E
    src_p = jnp.concatenate(
        [src, jnp.zeros((npadE,), jnp.int32)]).reshape(EPAD // K, K)
    dst_p = jnp.concatenate(
        [dst, jnp.full((npadE,), N, jnp.int32)]).reshape(EPAD // K, K)
    eidx = jnp.stack([src_p, dst_p], axis=1)
    x_p = jnp.pad(in_feat, ((0, NPAD - N), (0, 0)))
    zeros_tbl = jnp.zeros((NPAD, F), jnp.float32)

    b1r = b1.reshape(1, F)
    b2r = b2.reshape(1, F)
    bresr = bres.reshape(1, F)
    battnr = battn.reshape(1, 1)
    bf1r = bf1.reshape(1, F)
    bf2r = bf2.reshape(1, 1)
    b3r = b3.reshape(1, F)
    b4r = b4.reshape(1, 2)

    _deg_kernel, _segsum_kernel = _sc_kernels()
    deg_part = _deg_kernel(dst_p).reshape(NW, NPAD)
    h_pre, res = _mlp_call(x_p, W1, b1r, W2, b2r, Wres, bresr)
    dinv, s1 = _scale_call(deg_part, h_pre)
    part1 = _segsum_kernel(s1, eidx, zeros_tbl)
    f1, s2 = _update_call(part1, h_pre, dinv)
    part2 = _segsum_kernel(s2, eidx, zeros_tbl)
    logits_p = _final_call(part2, h_pre, f1, dinv, res,
                           Wattn, battnr, Wf1, bf1r, Wf2, bf2r,
                           W3, b3r, W4, b4r)
    return logits_p[:N]


# speedup vs baseline: 1.0175x; 1.0055x over previous
import functools

import jax
import jax.numpy as jnp
from jax import lax
from jax.experimental import pallas as pl
from jax.experimental.pallas import tpu as pltpu
from jax.experimental.pallas import tpu_sc as plsc

N = 10000
F = 128
E = 320000
NPAD = 10240
NC = 2
NS = 16
NW = NC * NS
K = 64
CHT = (-(-E // (NW * K)) + 7) // 8 * 8
EPAD = NW * K * CHT
RPT = NPAD // NS
RB = 2048
GRID = NPAD // RB


def _deg_body(dst_hbm, out_hbm, dstb, hist):
    cid = lax.axis_index("c")
    sid = lax.axis_index("s")
    wid = sid * NC + cid
    pltpu.sync_copy(dst_hbm.at[pl.ds(wid * CHT, CHT)], dstb)
    zero16 = jnp.zeros((16,), jnp.float32)

    def zbody(i, carry):
        hist[pl.ds(i * 16, 16)] = zero16
        return carry

    lax.fori_loop(0, NPAD // 16, zbody, 0)

    ones16 = jnp.ones((16,), jnp.float32)

    def ebody(i, carry):
        j = i // (K // 16)
        g = i % (K // 16)
        idx = dstb[j, pl.ds(g * 16, 16)]
        plsc.addupdate_scatter(hist, [idx], ones16)
        return carry

    lax.fori_loop(0, CHT * (K // 16), ebody, 0)
    pltpu.sync_copy(hist, out_hbm.at[wid, 0])


NBUF = 4
CHT_A = 296
CHT_B = (EPAD // K - 16 * CHT_A) // 16


def _seg_pipeline(tbl_hbm, eidx_hbm, acc, eb, rows, gsem, ssem, isem,
                  base, ng):
    pltpu.sync_copy(eidx_hbm.at[pl.ds(base, NBUF)], eb.at[0])
    pltpu.async_copy(eidx_hbm.at[pl.ds(base + NBUF, NBUF)], eb.at[1], isem)
    for b in range(NBUF):
        pltpu.async_copy(tbl_hbm.at[eb.at[0, b, 0]], rows[b], gsem[b])

    def group(g, carry):
        slot = lax.rem(g, 2)
        nslot = 1 - slot
        for b in range(NBUF):
            pltpu.make_async_copy(
                tbl_hbm.at[eb.at[slot, b, 0]], rows[b], gsem[b]).wait()
            pltpu.async_copy(
                rows[b], acc.at[eb.at[slot, b, 1]], ssem[b], add=True)

        @pl.when(g + 1 < ng)
        def _():
            pltpu.make_async_copy(
                eidx_hbm.at[pl.ds(base, NBUF)], eb.at[nslot], isem).wait()
            for b in range(NBUF):
                pltpu.make_async_copy(
                    rows[b], acc.at[eb.at[slot, b, 1]], ssem[b]).wait()
                pltpu.async_copy(
                    tbl_hbm.at[eb.at[nslot, b, 0]], rows[b], gsem[b])

        @pl.when(g + 2 < ng)
        def _():
            pltpu.async_copy(
                eidx_hbm.at[pl.ds(base + (g + 2) * NBUF, NBUF)],
                eb.at[slot], isem)
        return carry

    lax.fori_loop(0, ng, group, 0)
    for b in range(NBUF):
        pltpu.make_async_copy(rows[b], acc.at[eb.at[0, b, 1]], ssem[b]).wait()


def _segsum_body(tbl_hbm, eidx_hbm, out_hbm,
                 eb, r0, r1, r2, r3, acc,
                 g0, g1, g2, g3, s0, s1, s2, s3, isem):
    rows = [r0, r1, r2, r3]
    gsem = [g0, g1, g2, g3]
    ssem = [s0, s1, s2, s3]
    cid = lax.axis_index("c")
    sid = lax.axis_index("s")
    zero16 = jnp.zeros((16,), jnp.float32)

    def zbody(i, carry):
        rows[0][i // (F // 16), pl.ds((i % (F // 16)) * 16, 16)] = zero16
        return carry

    lax.fori_loop(0, (K * F) // 16, zbody, 0)
    for t in range(RPT // K):
        pltpu.sync_copy(rows[0], acc.at[pl.ds(sid * RPT + t * K, K)])
    plsc.subcore_barrier()

    @pl.when(cid == 0)
    def _():
        _seg_pipeline(tbl_hbm, eidx_hbm, acc, eb, rows, gsem, ssem, isem,
                      sid * CHT_A, CHT_A // NBUF)

    if CHT_B > 0:
        @pl.when(cid == 1)
        def _():
            _seg_pipeline(tbl_hbm, eidx_hbm, acc, eb, rows, gsem, ssem, isem,
                          16 * CHT_A + sid * CHT_B, CHT_B // NBUF)

    plsc.subcore_barrier()
    pltpu.sync_copy(acc.at[pl.ds(sid * RPT, RPT)],
                    out_hbm.at[cid, pl.ds(sid * RPT, RPT)])


@functools.lru_cache(maxsize=None)
def _sc_kernels():
    mesh = plsc.VectorSubcoreMesh(
        core_axis_name="c", subcore_axis_name="s",
        num_cores=NC, num_subcores=NS)
    cparams = pltpu.CompilerParams(needs_layout_passes=False)
    deg = pl.kernel(
        _deg_body,
        out_type=jax.ShapeDtypeStruct((NW, 1, NPAD), jnp.float32),
        mesh=mesh,
        compiler_params=cparams,
        scratch_types=[
            pltpu.VMEM((CHT, K), jnp.int32),
            pltpu.VMEM((NPAD,), jnp.float32),
        ],
    )
    segsum = pl.kernel(
        _segsum_body,
        out_type=jax.ShapeDtypeStruct((NC, NPAD, F), jnp.float32),
        mesh=mesh,
        compiler_params=cparams,
        scratch_types=(
            [pltpu.VMEM((2, NBUF, 2, K), jnp.int32)]
            + [pltpu.VMEM((K, F), jnp.float32) for _ in range(NBUF)]
            + [pltpu.VMEM_SHARED((NPAD, F), jnp.float32)]
            + [pltpu.SemaphoreType.DMA for _ in range(2 * NBUF + 1)]
        ),
    )
    return deg, segsum


def _mlp_body(x_ref, w1_ref, b1_ref, w2_ref, b2_ref, wr_ref, br_ref,
              h_ref, res_ref):
    x = x_ref[...]
    h1 = jnp.maximum(jnp.dot(x, w1_ref[...],
                             preferred_element_type=jnp.float32)
                     + b1_ref[...], 0.0)
    h = jnp.maximum(jnp.dot(h1, w2_ref[...],
                            preferred_element_type=jnp.float32)
                    + b2_ref[...], 0.0)
    h_ref[...] = h
    res_ref[...] = jnp.dot(h, wr_ref[...],
                           preferred_element_type=jnp.float32) + br_ref[...]


def _scale_body(degp_ref, h_ref, dinv_ref, s_ref):
    deg = jnp.sum(degp_ref[...], axis=0)
    dinv = lax.rsqrt(jnp.maximum(deg, 1.0))
    dinv_ref[...] = dinv
    s_ref[...] = h_ref[...] * dinv[:, None]


def _update_body(part_ref, h_ref, dinv_ref, f_ref, s_ref):
    agg = part_ref[0] + part_ref[1]
    dinv = dinv_ref[...][:, None]
    f = h_ref[...] - agg * dinv
    f_ref[...] = f
    s_ref[...] = f * dinv


def _final_body(part_ref, h_ref, f1_ref, dinv_ref, res_ref,
                wa_ref, ba_ref, wf1_ref, bf1_ref, wf2_ref, bf2_ref,
                w3_ref, b3_ref, w4_ref, b4_ref, out_ref):
    dinv = dinv_ref[...][:, None]
    f0 = h_ref[...]
    f1 = f1_ref[...]
    f2 = f1 - (part_ref[0] + part_ref[1]) * dinv
    br0 = 3.0 * f0 - 3.0 * f1 + 0.75 * f2
    br1 = 3.0 * f1 - 1.5 * f2
    br2 = 0.75 * f2
    wa = wa_ref[...]
    ba = ba_ref[...]
    s0 = jnp.dot(br0, wa, preferred_element_type=jnp.float32) + ba
    s1 = jnp.dot(br1, wa, preferred_element_type=jnp.float32) + ba
    s2 = jnp.dot(br2, wa, preferred_element_type=jnp.float32) + ba
    m = jnp.maximum(jnp.maximum(s0, s1), s2)
    e0 = jnp.exp(s0 - m)
    e1 = jnp.exp(s1 - m)
    e2 = jnp.exp(s2 - m)
    den = e0 + e1 + e2
    attn = (e0 * br0 + e1 * br1 + e2 * br2) / den
    mean = (br0 + br1 + br2) * (1.0 / 3.0)
    z = jnp.maximum(
        jnp.dot(attn, wf1_ref[0:F, :], preferred_element_type=jnp.float32)
        + jnp.dot(mean, wf1_ref[F:2 * F, :], preferred_element_type=jnp.float32)
        + bf1_ref[...], 0.0)
    logit_fw = jnp.dot(z, wf2_ref[...],
                       preferred_element_type=jnp.float32) + bf2_ref[...]
    fw = 1.0 / (1.0 + jnp.exp(-logit_fw))
    fused = 0.1 * fw * attn + (1.0 - fw) * mean + 0.8 * res_ref[...]
    h3 = jnp.maximum(jnp.dot(fused, w3_ref[...],
                             preferred_element_type=jnp.float32)
                     + b3_ref[...], 0.0)
    out_ref[...] = jnp.dot(h3, w4_ref[...],
                           preferred_element_type=jnp.float32) + b4_ref[...]


def _row_spec():
    return pl.BlockSpec((RB, F), lambda i: (i, 0))


def _full_spec(shape):
    nd = len(shape)
    return pl.BlockSpec(shape, lambda i: (0,) * nd)


def _mlp_call(x, W1, b1, W2, b2, Wres, bres):
    return pl.pallas_call(
        _mlp_body,
        grid=(GRID,),
        in_specs=[_row_spec(), _full_spec((F, F)), _full_spec((1, F)),
                  _full_spec((F, F)), _full_spec((1, F)),
                  _full_spec((F, F)), _full_spec((1, F))],
        out_specs=[_row_spec(), _row_spec()],
        out_shape=[jax.ShapeDtypeStruct((NPAD, F), jnp.float32),
                   jax.ShapeDtypeStruct((NPAD, F), jnp.float32)],
    )(x, W1, b1, W2, b2, Wres, bres)


def _scale_call(degp, h):
    return pl.pallas_call(
        _scale_body,
        grid=(GRID,),
        in_specs=[pl.BlockSpec((NW, RB), lambda i: (0, i)), _row_spec()],
        out_specs=[pl.BlockSpec((RB,), lambda i: (i,)), _row_spec()],
        out_shape=[jax.ShapeDtypeStruct((NPAD,), jnp.float32),
                   jax.ShapeDtypeStruct((NPAD, F), jnp.float32)],
    )(degp, h)


def _update_call(part, h, dinv):
    return pl.pallas_call(
        _update_body,
        grid=(GRID,),
        in_specs=[pl.BlockSpec((NC, RB, F), lambda i: (0, i, 0)),
                  _row_spec(), pl.BlockSpec((RB,), lambda i: (i,))],
        out_specs=[_row_spec(), _row_spec()],
        out_shape=[jax.ShapeDtypeStruct((NPAD, F), jnp.float32),
                   jax.ShapeDtypeStruct((NPAD, F), jnp.float32)],
    )(part, h, dinv)


def _final_call(part, h, f1, dinv, res,
                Wattn, battn, Wf1, bf1, Wf2, bf2, W3, b3, W4, b4):
    return pl.pallas_call(
        _final_body,
        grid=(GRID,),
        in_specs=[pl.BlockSpec((NC, RB, F), lambda i: (0, i, 0)),
                  _row_spec(), _row_spec(),
                  pl.BlockSpec((RB,), lambda i: (i,)), _row_spec(),
                  _full_spec((F, 1)), _full_spec((1, 1)),
                  _full_spec((2 * F, F)), _full_spec((1, F)),
                  _full_spec((F, 1)), _full_spec((1, 1)),
                  _full_spec((F, F)), _full_spec((1, F)),
                  _full_spec((F, 2)), _full_spec((1, 2))],
        out_specs=pl.BlockSpec((RB, 2), lambda i: (i, 0)),
        out_shape=jax.ShapeDtypeStruct((NPAD, 2), jnp.float32),
    )(part, h, f1, dinv, res,
      Wattn, battn, Wf1, bf1, Wf2, bf2, W3, b3, W4, b4)


def kernel(in_feat, edge_index, W1, b1, W2, b2, Wres, bres, Wattn, battn,
           Wf1, bf1, Wf2, bf2, W3, b3, W4, b4):
    src = edge_index[0].astype(jnp.int32)
    dst = edge_index[1].astype(jnp.int32)
    npadE = EPAD - E
    src_p = jnp.concatenate(
        [src, jnp.zeros((npadE,), jnp.int32)]).reshape(EPAD // K, K)
    dst_p = jnp.concatenate(
        [dst, jnp.full((npadE,), N, jnp.int32)]).reshape(EPAD // K, K)
    eidx = jnp.stack([src_p, dst_p], axis=1)
    x_p = jnp.pad(in_feat, ((0, NPAD - N), (0, 0)))

    b1r = b1.reshape(1, F)
    b2r = b2.reshape(1, F)
    bresr = bres.reshape(1, F)
    battnr = battn.reshape(1, 1)
    bf1r = bf1.reshape(1, F)
    bf2r = bf2.reshape(1, 1)
    b3r = b3.reshape(1, F)
    b4r = b4.reshape(1, 2)

    _deg_kernel, _segsum_kernel = _sc_kernels()
    deg_part = _deg_kernel(dst_p).reshape(NW, NPAD)
    h_pre, res = _mlp_call(x_p, W1, b1r, W2, b2r, Wres, bresr)
    dinv, s1 = _scale_call(deg_part, h_pre)
    part1 = _segsum_kernel(s1, eidx)
    f1, s2 = _update_call(part1, h_pre, dinv)
    part2 = _segsum_kernel(s2, eidx)
    logits_p = _final_call(part2, h_pre, f1, dinv, res,
                           Wattn, battnr, Wf1, bf1r, Wf2, bf2r,
                           W3, b3r, W4, b4r)
    return logits_p[:N]
